# baseline (speedup 1.0000x reference)
import functools

import jax
import jax.numpy as jnp
from jax import lax
from jax.experimental import pallas as pl
from jax.experimental.pallas import tpu as pltpu
from jax.experimental.pallas import tpu_sc as plsc

B = 4
N = 16384
G = 512
M = 32
NR = 128
NC = 128
GR = 4

_BIG = 1 << 30


def _fps_body(xt_ref, craw_ref):
    x = xt_ref[0, 0]
    y = xt_ref[0, 1]
    z = xt_ref[0, 2]
    fiota = (lax.broadcasted_iota(jnp.int32, (NR, NC), 0) * NC
             + lax.broadcasted_iota(jnp.int32, (NR, NC), 1))
    giota = (lax.broadcasted_iota(jnp.int32, (GR, 128), 0) * 128
             + lax.broadcasted_iota(jnp.int32, (GR, 128), 1))

    def extract(oh):
        px = jnp.sum(jnp.where(oh, x, 0.0))
        py = jnp.sum(jnp.where(oh, y, 0.0))
        pz = jnp.sum(jnp.where(oh, z, 0.0))
        return px, py, pz

    px0, py0, pz0 = extract(fiota == 0)
    zero = jnp.zeros((GR, 128), dtype=jnp.float32)
    dists0 = jnp.full((NR, NC), jnp.inf, dtype=jnp.float32)

    def step(g, carry):
        px, py, pz, dists, cx, cy, cz = carry
        oh_g = giota == g
        cx = jnp.where(oh_g, px, cx)
        cy = jnp.where(oh_g, py, cy)
        cz = jnp.where(oh_g, pz, cz)
        dx = x - px
        dy = y - py
        dz = z - pz
        d = dx * dx + dy * dy + dz * dz
        dists = jnp.minimum(dists, d)
        m = jnp.max(dists)
        sel = jnp.where(dists == m, fiota, _BIG)
        i = jnp.min(sel)
        npx, npy, npz = extract(fiota == i)
        return npx, npy, npz, dists, cx, cy, cz

    carry = (px0, py0, pz0, dists0, zero, zero, zero)
    carry = lax.fori_loop(0, G, step, carry)
    _, _, _, _, cx, cy, cz = carry
    craw_ref[0, 0] = cx
    craw_ref[0, 1] = cy
    craw_ref[0, 2] = cz


def _fps_call(xt, interpret=False):
    return pl.pallas_call(
        _fps_body,
        grid=(B,),
        in_specs=[pl.BlockSpec((1, 3, NR, NC), lambda b: (b, 0, 0, 0))],
        out_specs=pl.BlockSpec((1, 3, GR, 128), lambda b: (b, 0, 0, 0)),
        out_shape=jax.ShapeDtypeStruct((B, 3, GR, 128), jnp.float32),
        interpret=interpret,
    )(xt)


CPB = 8


def _knn_body(centers_ref, xt_ref, idx_ref):
    x = xt_ref[0, 0]
    y = xt_ref[0, 1]
    z = xt_ref[0, 2]
    fiota = (lax.broadcasted_iota(jnp.int32, (NR, NC), 0) * NC
             + lax.broadcasted_iota(jnp.int32, (NR, NC), 1))
    inf = jnp.float32(jnp.inf)
    for c in range(CPB):
        cx = centers_ref[0, c, 0]
        cy = centers_ref[0, c, 1]
        cz = centers_ref[0, c, 2]
        dx = cx - x
        dy = cy - y
        dz = cz - z
        d2 = dx * dx + dy * dy + dz * dz
        kiota = lax.broadcasted_iota(jnp.int32, (1, M), 1)
        row = jnp.zeros((1, M), dtype=jnp.int32)
        for k in range(M):
            m = jnp.min(d2)
            sel = jnp.where(d2 == m, fiota, _BIG)
            i = jnp.min(sel)
            row = jnp.where(kiota == k, i, row)
            d2 = jnp.where(fiota == i, inf, d2)
        idx_ref[0, pl.ds(c, 1), :] = row


def _knn_call(centers, xt, interpret=False):
    return pl.pallas_call(
        _knn_body,
        grid=(B, G // CPB),
        in_specs=[
            pl.BlockSpec((1, CPB, 3), lambda b, j: (b, j, 0),
                         memory_space=pltpu.SMEM),
            pl.BlockSpec((1, 3, NR, NC), lambda b, j: (b, 0, 0, 0)),
        ],
        out_specs=pl.BlockSpec((1, CPB, M), lambda b, j: (b * (G // CPB) + j, 0, 0)),
        out_shape=jax.ShapeDtypeStruct((B * G // CPB, CPB, M), jnp.int32),
        interpret=interpret,
    )(centers, xt)


_K3_GATHER = True
NW = 32
WPB = NW // B
RPW = G // WPB
EPW = RPW * M


def _gather_body(xt_hbm, ct_hbm, idx_hbm, out_hbm, xv, yv, zv, cxv, cyv, czv,
                 iv, ov):
    w = lax.axis_index("s") * 2 + lax.axis_index("c")
    b = w // WPB
    r = w % WPB
    pltpu.sync_copy(xt_hbm.at[pl.ds(b * 3 * N, N)], xv)
    pltpu.sync_copy(xt_hbm.at[pl.ds((b * 3 + 1) * N, N)], yv)
    pltpu.sync_copy(xt_hbm.at[pl.ds((b * 3 + 2) * N, N)], zv)
    pltpu.sync_copy(ct_hbm.at[pl.ds(b * 3 * G, G)], cxv)
    pltpu.sync_copy(ct_hbm.at[pl.ds((b * 3 + 1) * G, G)], cyv)
    pltpu.sync_copy(ct_hbm.at[pl.ds((b * 3 + 2) * G, G)], czv)
    off = b * G * M + r * EPW
    pltpu.sync_copy(idx_hbm.at[pl.ds(off, EPW)], iv)
    gbase = r * RPW

    def body(t, _):
        lane = lax.iota(jnp.int32, 16)
        ii = iv[pl.ds(t * 16, 16)]
        ig = gbase + (t * 16 + lane) // M
        gx = plsc.load_gather(xv, [ii])
        gy = plsc.load_gather(yv, [ii])
        gz = plsc.load_gather(zv, [ii])
        hx = plsc.load_gather(cxv, [ig])
        hy = plsc.load_gather(cyv, [ig])
        hz = plsc.load_gather(czv, [ig])
        ov[pl.ds(t * 16, 16)] = gx - hx
        ov[pl.ds(EPW + t * 16, 16)] = gy - hy
        ov[pl.ds(2 * EPW + t * 16, 16)] = gz - hz
        return _

    if _K3_GATHER:
        lax.fori_loop(0, EPW // 16, body, 0)
    pltpu.sync_copy(ov.at[pl.ds(0, EPW)], out_hbm.at[pl.ds(0 * B * G * M + off, EPW)])
    pltpu.sync_copy(ov.at[pl.ds(EPW, EPW)], out_hbm.at[pl.ds(1 * B * G * M + off, EPW)])
    pltpu.sync_copy(ov.at[pl.ds(2 * EPW, EPW)], out_hbm.at[pl.ds(2 * B * G * M + off, EPW)])


def _gather_call(xt_flat, ct_flat, idx_flat):
    mesh = plsc.VectorSubcoreMesh(core_axis_name="c", subcore_axis_name="s")
    kfn = pl.kernel(
        _gather_body,
        mesh=mesh,
        compiler_params=pltpu.CompilerParams(needs_layout_passes=False),
        out_type=jax.ShapeDtypeStruct((3 * B * G * M,), jnp.float32),
        scratch_types=[
            pltpu.VMEM((N,), jnp.float32),
            pltpu.VMEM((N,), jnp.float32),
            pltpu.VMEM((N,), jnp.float32),
            pltpu.VMEM((G,), jnp.float32),
            pltpu.VMEM((G,), jnp.float32),
            pltpu.VMEM((G,), jnp.float32),
            pltpu.VMEM((EPW,), jnp.int32),
            pltpu.VMEM((3 * EPW,), jnp.float32),
        ],
    )
    return kfn(xt_flat, ct_flat, idx_flat)


@jax.jit
def kernel(xyz):
    xt = jnp.transpose(xyz, (0, 2, 1)).reshape(B, 3, NR, NC)
    craw = _fps_call(xt)
    centers = jnp.transpose(craw.reshape(B, 3, G), (0, 2, 1))
    idx = _knn_call(centers, xt).reshape(B, G, M)
    out3 = _gather_call(
        xt.reshape(B * 3 * N),
        craw.reshape(B * 3 * G),
        idx.reshape(B * G * M),
    )
    neighborhood = jnp.transpose(out3.reshape(3, B, G, M), (1, 2, 3, 0))
    return neighborhood, centers


# speedup vs baseline: 7.8262x; 7.8262x over previous
import jax
import jax.numpy as jnp
from jax import lax
from jax.experimental import pallas as pl
from jax.experimental.pallas import tpu as pltpu
from jax.experimental.pallas import tpu_sc as plsc

B = 4
N = 16384
G = 512
M = 32
NR = 128
NC = 128
GR = 4

_BIG = 1 << 30


def _fps_body(xt_ref, craw_ref):
    x = xt_ref[0, 0]
    y = xt_ref[0, 1]
    z = xt_ref[0, 2]
    fiota = (lax.broadcasted_iota(jnp.int32, (NR, NC), 0) * NC
             + lax.broadcasted_iota(jnp.int32, (NR, NC), 1))
    giota = (lax.broadcasted_iota(jnp.int32, (GR, 128), 0) * 128
             + lax.broadcasted_iota(jnp.int32, (GR, 128), 1))

    def extract(oh):
        px = jnp.sum(jnp.where(oh, x, 0.0))
        py = jnp.sum(jnp.where(oh, y, 0.0))
        pz = jnp.sum(jnp.where(oh, z, 0.0))
        return px, py, pz

    px0, py0, pz0 = extract(fiota == 0)
    zero = jnp.zeros((GR, 128), dtype=jnp.float32)
    dists0 = jnp.full((NR, NC), jnp.inf, dtype=jnp.float32)

    def step(g, carry):
        px, py, pz, dists, cx, cy, cz = carry
        oh_g = giota == g
        cx = jnp.where(oh_g, px, cx)
        cy = jnp.where(oh_g, py, cy)
        cz = jnp.where(oh_g, pz, cz)
        dx = x - px
        dy = y - py
        dz = z - pz
        d = dx * dx + dy * dy + dz * dz
        dists = jnp.minimum(dists, d)
        m = jnp.max(dists)
        sel = jnp.where(dists == m, fiota, _BIG)
        i = jnp.min(sel)
        npx, npy, npz = extract(fiota == i)
        return npx, npy, npz, dists, cx, cy, cz

    carry = (px0, py0, pz0, dists0, zero, zero, zero)
    carry = lax.fori_loop(0, G, step, carry)
    _, _, _, _, cx, cy, cz = carry
    craw_ref[0, 0] = cx
    craw_ref[0, 1] = cy
    craw_ref[0, 2] = cz


def _fps_call(xt, interpret=False):
    return pl.pallas_call(
        _fps_body,
        grid=(B,),
        in_specs=[pl.BlockSpec((1, 3, NR, NC), lambda b: (b, 0, 0, 0))],
        out_specs=pl.BlockSpec((1, 3, GR, 128), lambda b: (b, 0, 0, 0)),
        out_shape=jax.ShapeDtypeStruct((B, 3, GR, 128), jnp.float32),
        interpret=interpret,
    )(xt)


CPB = 8


def _d2_body(centers_ref, xt_ref, d2_ref, t_ref):
    x = xt_ref[0, 0]
    y = xt_ref[0, 1]
    z = xt_ref[0, 2]
    siota = lax.broadcasted_iota(jnp.int32, (NR, 1), 0)
    neg_inf = jnp.float32(-jnp.inf)
    for c in range(CPB):
        cx = centers_ref[0, c, 0]
        cy = centers_ref[0, c, 1]
        cz = centers_ref[0, c, 2]
        dx = cx - x
        dy = cy - y
        dz = cz - z
        d2 = dx * dx + dy * dy + dz * dz
        d2_ref[c] = d2
        rm = jnp.min(d2, axis=1, keepdims=True)
        r1 = jnp.minimum(rm, pltpu.roll(rm, NR - 1, 0))
        r2 = jnp.minimum(r1, pltpu.roll(r1, NR - 2, 0))
        t = jnp.max(jnp.where(siota % 4 == 0, r2, neg_inf))
        t_ref[0, 0, c] = t


def _d2_call(centers, xt, interpret=False):
    return pl.pallas_call(
        _d2_body,
        grid=(B, G // CPB),
        in_specs=[
            pl.BlockSpec((1, CPB, 3), lambda b, j: (b, j, 0),
                         memory_space=pltpu.SMEM),
            pl.BlockSpec((1, 3, NR, NC), lambda b, j: (b, 0, 0, 0)),
        ],
        out_specs=[
            pl.BlockSpec((CPB, NR, NC), lambda b, j: (b * (G // CPB) + j, 0, 0)),
            pl.BlockSpec((1, 1, CPB), lambda b, j: (b * (G // CPB) + j, 0, 0),
                         memory_space=pltpu.SMEM),
        ],
        out_shape=[
            jax.ShapeDtypeStruct((B * G, NR, NC), jnp.float32),
            jax.ShapeDtypeStruct((B * G // CPB, 1, CPB), jnp.float32),
        ],
        interpret=interpret,
    )(centers, xt)


NW = 32
WPB = NW // B
RPW = G // WPB
EPW = RPW * M
CAP = 1024


def _sel_body(xt_hbm, ct_hbm, t_hbm, d2_hbm, out_hbm,
              xv, yv, zv, cxv, cyv, czv, tv, dv, candv, candi, ov):
    w = lax.axis_index("s") * 2 + lax.axis_index("c")
    b = w // WPB
    r = w % WPB
    pltpu.sync_copy(xt_hbm.at[pl.ds(b * 3 * N, N)], xv)
    pltpu.sync_copy(xt_hbm.at[pl.ds((b * 3 + 1) * N, N)], yv)
    pltpu.sync_copy(xt_hbm.at[pl.ds((b * 3 + 2) * N, N)], zv)
    pltpu.sync_copy(ct_hbm.at[pl.ds(b * 3 * G, G)], cxv)
    pltpu.sync_copy(ct_hbm.at[pl.ds((b * 3 + 1) * G, G)], cyv)
    pltpu.sync_copy(ct_hbm.at[pl.ds((b * 3 + 2) * G, G)], czv)
    row0 = b * G + r * RPW
    pltpu.sync_copy(t_hbm.at[pl.ds(row0, RPW)], tv)

    lane = lax.iota(jnp.int32, 16)
    inf16 = jnp.full((16,), jnp.inf, dtype=jnp.float32)
    big16 = jnp.full((16,), _BIG, dtype=jnp.int32)

    def row_body(q, _):
        pltpu.sync_copy(d2_hbm.at[pl.ds((row0 + q) * N, N)], dv)
        tsv = plsc.load_gather(tv, [jnp.full((16,), q, dtype=jnp.int32)])

        def chunk(t, off):
            v = dv[pl.ds(t * 16, 16)]
            mask = v <= tsv

            def take(off):
                plsc.store_compressed(candv.at[pl.ds(off, 16)], v, mask=mask)
                ivec = t * 16 + lane
                plsc.store_compressed(candi.at[pl.ds(off, 16)], ivec,
                                      mask=mask)
                cnt = jnp.sum(mask.astype(jnp.int32))
                return jnp.minimum(off + cnt, CAP - 16)

            return lax.cond(jnp.any(mask), take, lambda off: off, off)

        off = lax.fori_loop(0, N // 16, chunk, jnp.int32(0))
        candv[pl.ds(off, 16)] = inf16
        candi[pl.ds(off, 16)] = big16
        nv = off // 16 + 1

        def select(k, carry):
            mprev, iprev, sel0, sel1 = carry

            def pass1(t, mv):
                cv = candv[pl.ds(t * 16, 16)]
                ci = candi[pl.ds(t * 16, 16)]
                elig = (cv > mprev) | ((cv == mprev) & (ci > iprev))
                return jnp.minimum(mv, jnp.where(elig, cv, inf16))

            m = jnp.min(lax.fori_loop(0, nv, pass1, inf16))

            def pass2(t, iv):
                cv = candv[pl.ds(t * 16, 16)]
                ci = candi[pl.ds(t * 16, 16)]
                elig = (cv == m) & ((cv > mprev) | (ci > iprev))
                return jnp.minimum(iv, jnp.where(elig, ci, big16))

            i = jnp.min(lax.fori_loop(0, nv, pass2, big16))
            sel0 = jnp.where(lane == k, i, sel0)
            sel1 = jnp.where(lane == (k - 16), i, sel1)
            return m, i, sel0, sel1

        zero16 = jnp.zeros((16,), dtype=jnp.int32)
        _, _, sel0, sel1 = lax.fori_loop(
            0, M, select, (jnp.float32(-jnp.inf), jnp.int32(-1),
                           zero16, zero16))

        gl = jnp.full((16,), r * RPW + q, dtype=jnp.int32)
        hx = plsc.load_gather(cxv, [gl])
        hy = plsc.load_gather(cyv, [gl])
        hz = plsc.load_gather(czv, [gl])
        o = q * M
        ov[pl.ds(o, 16)] = plsc.load_gather(xv, [sel0]) - hx
        ov[pl.ds(o + 16, 16)] = plsc.load_gather(xv, [sel1]) - hx
        ov[pl.ds(EPW + o, 16)] = plsc.load_gather(yv, [sel0]) - hy
        ov[pl.ds(EPW + o + 16, 16)] = plsc.load_gather(yv, [sel1]) - hy
        ov[pl.ds(2 * EPW + o, 16)] = plsc.load_gather(zv, [sel0]) - hz
        ov[pl.ds(2 * EPW + o + 16, 16)] = plsc.load_gather(zv, [sel1]) - hz
        return _

    lax.fori_loop(0, RPW, row_body, 0)
    off_out = b * G * M + r * EPW
    pltpu.sync_copy(ov.at[pl.ds(0, EPW)],
                    out_hbm.at[pl.ds(0 * B * G * M + off_out, EPW)])
    pltpu.sync_copy(ov.at[pl.ds(EPW, EPW)],
                    out_hbm.at[pl.ds(1 * B * G * M + off_out, EPW)])
    pltpu.sync_copy(ov.at[pl.ds(2 * EPW, EPW)],
                    out_hbm.at[pl.ds(2 * B * G * M + off_out, EPW)])


def _sel_call(xt_flat, ct_flat, t_flat, d2_flat):
    mesh = plsc.VectorSubcoreMesh(core_axis_name="c", subcore_axis_name="s")
    kfn = pl.kernel(
        _sel_body,
        mesh=mesh,
        compiler_params=pltpu.CompilerParams(needs_layout_passes=False),
        out_type=jax.ShapeDtypeStruct((3 * B * G * M,), jnp.float32),
        scratch_types=[
            pltpu.VMEM((N,), jnp.float32),
            pltpu.VMEM((N,), jnp.float32),
            pltpu.VMEM((N,), jnp.float32),
            pltpu.VMEM((G,), jnp.float32),
            pltpu.VMEM((G,), jnp.float32),
            pltpu.VMEM((G,), jnp.float32),
            pltpu.VMEM((RPW,), jnp.float32),
            pltpu.VMEM((N,), jnp.float32),
            pltpu.VMEM((CAP,), jnp.float32),
            pltpu.VMEM((CAP,), jnp.int32),
            pltpu.VMEM((3 * EPW,), jnp.float32),
        ],
    )
    return kfn(xt_flat, ct_flat, t_flat, d2_flat)


@jax.jit
def kernel(xyz):
    xt = jnp.transpose(xyz, (0, 2, 1)).reshape(B, 3, NR, NC)
    craw = _fps_call(xt)
    centers = jnp.transpose(craw.reshape(B, 3, G), (0, 2, 1))
    d2, tthr = _d2_call(centers, xt)
    out3 = _sel_call(
        xt.reshape(B * 3 * N),
        craw.reshape(B * 3 * G),
        tthr.reshape(B * G),
        d2.reshape(B * G * N),
    )
    neighborhood = jnp.transpose(out3.reshape(3, B, G, M), (1, 2, 3, 0))
    return neighborhood, centers


# speedup vs baseline: 14.3485x; 1.8334x over previous
import jax
import jax.numpy as jnp
from jax import lax
from jax.experimental import pallas as pl
from jax.experimental.pallas import tpu as pltpu
from jax.experimental.pallas import tpu_sc as plsc

B = 4
N = 16384
G = 512
M = 32
NR = 128
NC = 128

_BIG = 1 << 30


def _fps_body(xs_ref, xt_ref, cs_ref):
    x = xt_ref[0, 0]
    y = xt_ref[0, 1]
    z = xt_ref[0, 2]
    fiota = (lax.broadcasted_iota(jnp.int32, (NR, NC), 0) * NC
             + lax.broadcasted_iota(jnp.int32, (NR, NC), 1))

    px0 = xs_ref[0, 0, 0]
    py0 = xs_ref[0, 1, 0]
    pz0 = xs_ref[0, 2, 0]
    dists0 = jnp.full((NR, NC), jnp.inf, dtype=jnp.float32)

    def step(g, carry):
        px, py, pz, dists = carry
        cs_ref[0, 0, g] = px
        cs_ref[0, 1, g] = py
        cs_ref[0, 2, g] = pz
        dx = x - px
        dy = y - py
        dz = z - pz
        d = dx * dx + dy * dy + dz * dz
        dists = jnp.minimum(dists, d)
        m = jnp.max(dists)
        sel = jnp.where(dists == m, fiota, _BIG)
        i = jnp.min(sel)
        npx = xs_ref[0, 0, i]
        npy = xs_ref[0, 1, i]
        npz = xs_ref[0, 2, i]
        return npx, npy, npz, dists

    lax.fori_loop(0, G, step, (px0, py0, pz0, dists0))


def _fps_call(xflat, xt, interpret=False):
    return pl.pallas_call(
        _fps_body,
        grid=(B,),
        in_specs=[
            pl.BlockSpec((1, 3, N), lambda b: (b, 0, 0),
                         memory_space=pltpu.SMEM),
            pl.BlockSpec((1, 3, NR, NC), lambda b: (b, 0, 0, 0)),
        ],
        out_specs=pl.BlockSpec((1, 3, G), lambda b: (b, 0, 0),
                               memory_space=pltpu.SMEM),
        out_shape=jax.ShapeDtypeStruct((B, 3, G), jnp.float32),
        interpret=interpret,
    )(xflat, xt)


CPB = 8


def _d2_body(centers_ref, xt_ref, d2_ref, t_ref, rm_ref):
    x = xt_ref[0, 0]
    y = xt_ref[0, 1]
    z = xt_ref[0, 2]
    ident = (lax.broadcasted_iota(jnp.int32, (NR, NR), 0)
             == lax.broadcasted_iota(jnp.int32, (NR, NR), 1)
             ).astype(jnp.float32)
    neg_inf = jnp.float32(-jnp.inf)
    for c in range(CPB):
        cx = centers_ref[0, c, 0]
        cy = centers_ref[0, c, 1]
        cz = centers_ref[0, c, 2]
        dx = cx - x
        dy = cy - y
        dz = cz - z
        d2 = dx * dx + dy * dy + dz * dz
        d2_ref[c] = d2
        rm = jnp.min(d2, axis=1, keepdims=True)
        rmt = lax.dot_general(rm, ident, (((0,), (0,)), ((), ())),
                              precision=lax.Precision.HIGHEST)
        rank = jnp.sum((rmt < rm).astype(jnp.int32), axis=1, keepdims=True)
        t = jnp.max(jnp.where(rank < M, rm, neg_inf))
        t_ref[0, 0, c] = t
        rm_ref[c] = rmt


def _d2_call(centers, xt, interpret=False):
    return pl.pallas_call(
        _d2_body,
        grid=(B, G // CPB),
        in_specs=[
            pl.BlockSpec((1, CPB, 3), lambda b, j: (b, j, 0),
                         memory_space=pltpu.SMEM),
            pl.BlockSpec((1, 3, NR, NC), lambda b, j: (b, 0, 0, 0)),
        ],
        out_specs=[
            pl.BlockSpec((CPB, NR, NC), lambda b, j: (b * (G // CPB) + j, 0, 0)),
            pl.BlockSpec((1, 1, CPB), lambda b, j: (b * (G // CPB) + j, 0, 0),
                         memory_space=pltpu.SMEM),
            pl.BlockSpec((CPB, 1, NR), lambda b, j: (b * (G // CPB) + j, 0, 0)),
        ],
        out_shape=[
            jax.ShapeDtypeStruct((B * G, NR, NC), jnp.float32),
            jax.ShapeDtypeStruct((B * G // CPB, 1, CPB), jnp.float32),
            jax.ShapeDtypeStruct((B * G, 1, NR), jnp.float32),
        ],
        interpret=interpret,
    )(centers, xt)


NW = 32
WPB = NW // B
RPW = G // WPB
EPW = RPW * M
CAP = 1024


def _sel_body(xt_hbm, ct_hbm, t_hbm, d2_hbm, rm_hbm, out_hbm,
              xv, yv, zv, cxv, cyv, czv, tv, rmv, blist, dv,
              candv, candi, ov):
    w = lax.axis_index("s") * 2 + lax.axis_index("c")
    b = w // WPB
    r = w % WPB
    pltpu.sync_copy(xt_hbm.at[pl.ds(b * 3 * N, N)], xv)
    pltpu.sync_copy(xt_hbm.at[pl.ds((b * 3 + 1) * N, N)], yv)
    pltpu.sync_copy(xt_hbm.at[pl.ds((b * 3 + 2) * N, N)], zv)
    pltpu.sync_copy(ct_hbm.at[pl.ds(b * 3 * G, G)], cxv)
    pltpu.sync_copy(ct_hbm.at[pl.ds((b * 3 + 1) * G, G)], cyv)
    pltpu.sync_copy(ct_hbm.at[pl.ds((b * 3 + 2) * G, G)], czv)
    row0 = b * G + r * RPW
    pltpu.sync_copy(t_hbm.at[pl.ds(row0, RPW)], tv)

    lane = lax.iota(jnp.int32, 16)
    inf16 = jnp.full((16,), jnp.inf, dtype=jnp.float32)
    big16 = jnp.full((16,), _BIG, dtype=jnp.int32)

    def row_body(q, _):
        pltpu.sync_copy(d2_hbm.at[pl.ds((row0 + q) * N, N)], dv)
        pltpu.sync_copy(rm_hbm.at[pl.ds((row0 + q) * NR, NR)], rmv)
        tsv = plsc.load_gather(tv, [jnp.full((16,), q, dtype=jnp.int32)])

        def fchunk(t, off):
            rv = rmv[pl.ds(t * 16, 16)]
            mask = rv <= tsv
            plsc.store_compressed(blist.at[pl.ds(off, 16)],
                                  t * 16 + lane, mask=mask)
            return off + jnp.sum(mask.astype(jnp.int32))

        nb = lax.fori_loop(0, NR // 16, fchunk, jnp.int32(0))

        def rchunk(u, carry):
            off, tcnt = carry
            rvec = plsc.load_gather(
                blist, [jnp.full((16,), u, dtype=jnp.int32)])
            base = jnp.sum(jnp.where(lane == 0, rvec, 0)) * NC

            def ichunk(t, c2):
                off2, tc2 = c2
                v = dv[pl.ds(base + t * 16, 16)]
                mask = v <= tsv
                plsc.store_compressed(candv.at[pl.ds(off2, 16)], v,
                                      mask=mask)
                plsc.store_compressed(candi.at[pl.ds(off2, 16)],
                                      base + t * 16 + lane, mask=mask)
                cnt = jnp.sum(mask.astype(jnp.int32))
                return jnp.minimum(off2 + cnt, CAP - 16), tc2 + cnt

            return lax.fori_loop(0, NC // 16, ichunk, (off, tcnt))

        off, tcnt = lax.fori_loop(0, nb, rchunk,
                                  (jnp.int32(0), jnp.int32(0)))
        candv[pl.ds(off, 16)] = inf16
        candi[pl.ds(off, 16)] = big16
        nv = off // 16 + 1

        def run_select(load_pair, nvec):
            def select(k, carry):
                mprev, iprev, sel0, sel1 = carry

                def pass1(t, mv):
                    cv, ci = load_pair(t)
                    elig = (cv > mprev) | ((cv == mprev) & (ci > iprev))
                    return jnp.minimum(mv, jnp.where(elig, cv, inf16))

                m = jnp.min(lax.fori_loop(0, nvec, pass1, inf16))

                def pass2(t, iv):
                    cv, ci = load_pair(t)
                    elig = (cv == m) & ((cv > mprev) | (ci > iprev))
                    return jnp.minimum(iv, jnp.where(elig, ci, big16))

                i = jnp.min(lax.fori_loop(0, nvec, pass2, big16))
                sel0 = jnp.where(lane == k, i, sel0)
                sel1 = jnp.where(lane == (k - 16), i, sel1)
                return m, i, sel0, sel1

            zero16 = jnp.zeros((16,), dtype=jnp.int32)
            _, _, sel0, sel1 = lax.fori_loop(
                0, M, select, (jnp.float32(-jnp.inf), jnp.int32(-1),
                               zero16, zero16))
            return sel0, sel1

        def load_cand(t):
            return candv[pl.ds(t * 16, 16)], candi[pl.ds(t * 16, 16)]

        def load_full(t):
            return dv[pl.ds(t * 16, 16)], t * 16 + lane

        sel0, sel1 = lax.cond(
            tcnt <= CAP - 16,
            lambda: run_select(load_cand, nv),
            lambda: run_select(load_full, jnp.int32(N // 16)))

        gl = jnp.full((16,), r * RPW + q, dtype=jnp.int32)
        hx = plsc.load_gather(cxv, [gl])
        hy = plsc.load_gather(cyv, [gl])
        hz = plsc.load_gather(czv, [gl])
        o = q * M
        ov[pl.ds(o, 16)] = plsc.load_gather(xv, [sel0]) - hx
        ov[pl.ds(o + 16, 16)] = plsc.load_gather(xv, [sel1]) - hx
        ov[pl.ds(EPW + o, 16)] = plsc.load_gather(yv, [sel0]) - hy
        ov[pl.ds(EPW + o + 16, 16)] = plsc.load_gather(yv, [sel1]) - hy
        ov[pl.ds(2 * EPW + o, 16)] = plsc.load_gather(zv, [sel0]) - hz
        ov[pl.ds(2 * EPW + o + 16, 16)] = plsc.load_gather(zv, [sel1]) - hz
        return _

    lax.fori_loop(0, RPW, row_body, 0)
    off_out = b * G * M + r * EPW
    pltpu.sync_copy(ov.at[pl.ds(0, EPW)],
                    out_hbm.at[pl.ds(0 * B * G * M + off_out, EPW)])
    pltpu.sync_copy(ov.at[pl.ds(EPW, EPW)],
                    out_hbm.at[pl.ds(1 * B * G * M + off_out, EPW)])
    pltpu.sync_copy(ov.at[pl.ds(2 * EPW, EPW)],
                    out_hbm.at[pl.ds(2 * B * G * M + off_out, EPW)])


def _sel_call(xt_flat, ct_flat, t_flat, d2_flat, rm_flat):
    mesh = plsc.VectorSubcoreMesh(core_axis_name="c", subcore_axis_name="s")
    kfn = pl.kernel(
        _sel_body,
        mesh=mesh,
        compiler_params=pltpu.CompilerParams(needs_layout_passes=False),
        out_type=jax.ShapeDtypeStruct((3 * B * G * M,), jnp.float32),
        scratch_types=[
            pltpu.VMEM((N,), jnp.float32),
            pltpu.VMEM((N,), jnp.float32),
            pltpu.VMEM((N,), jnp.float32),
            pltpu.VMEM((G,), jnp.float32),
            pltpu.VMEM((G,), jnp.float32),
            pltpu.VMEM((G,), jnp.float32),
            pltpu.VMEM((RPW,), jnp.float32),
            pltpu.VMEM((NR,), jnp.float32),
            pltpu.VMEM((NR + 16,), jnp.int32),
            pltpu.VMEM((N,), jnp.float32),
            pltpu.VMEM((CAP,), jnp.float32),
            pltpu.VMEM((CAP,), jnp.int32),
            pltpu.VMEM((3 * EPW,), jnp.float32),
        ],
    )
    return kfn(xt_flat, ct_flat, t_flat, d2_flat, rm_flat)


@jax.jit
def kernel(xyz):
    xt = jnp.transpose(xyz, (0, 2, 1)).reshape(B, 3, NR, NC)
    cs = _fps_call(xt.reshape(B, 3, N), xt)
    centers = jnp.transpose(cs, (0, 2, 1))
    d2, tthr, rmt = _d2_call(centers, xt)
    out3 = _sel_call(
        xt.reshape(B * 3 * N),
        cs.reshape(B * 3 * G),
        tthr.reshape(B * G),
        d2.reshape(B * G * N),
        rmt.reshape(B * G * NR),
    )
    neighborhood = jnp.transpose(out3.reshape(3, B, G, M), (1, 2, 3, 0))
    return neighborhood, centers


# speedup vs baseline: 14.3724x; 1.0017x over previous
import jax
import jax.numpy as jnp
from jax import lax
from jax.experimental import pallas as pl
from jax.experimental.pallas import tpu as pltpu
from jax.experimental.pallas import tpu_sc as plsc

B = 4
N = 16384
G = 512
M = 32
NR = 128
NC = 128

_BIG = 1 << 30


def _fps_body(xs_ref, xt_ref, cs_ref):
    x = xt_ref[0, 0]
    y = xt_ref[0, 1]
    z = xt_ref[0, 2]
    fiota = (lax.broadcasted_iota(jnp.int32, (NR, NC), 0) * NC
             + lax.broadcasted_iota(jnp.int32, (NR, NC), 1))

    px0 = xs_ref[0, 0, 0]
    py0 = xs_ref[0, 1, 0]
    pz0 = xs_ref[0, 2, 0]
    dists0 = jnp.full((NR, NC), jnp.inf, dtype=jnp.float32)

    def step(g, carry):
        px, py, pz, dists = carry
        cs_ref[0, 0, g] = px
        cs_ref[0, 1, g] = py
        cs_ref[0, 2, g] = pz
        dx = x - px
        dy = y - py
        dz = z - pz
        d = dx * dx + dy * dy + dz * dz
        dists = jnp.minimum(dists, d)
        m = jnp.max(dists)
        sel = jnp.where(dists == m, fiota, _BIG)
        i = jnp.min(sel)
        npx = xs_ref[0, 0, i]
        npy = xs_ref[0, 1, i]
        npz = xs_ref[0, 2, i]
        return npx, npy, npz, dists

    lax.fori_loop(0, G, step, (px0, py0, pz0, dists0))


def _fps_call(xflat, xt, interpret=False):
    return pl.pallas_call(
        _fps_body,
        grid=(B,),
        in_specs=[
            pl.BlockSpec((1, 3, N), lambda b: (b, 0, 0),
                         memory_space=pltpu.SMEM),
            pl.BlockSpec((1, 3, NR, NC), lambda b: (b, 0, 0, 0)),
        ],
        out_specs=pl.BlockSpec((1, 3, G), lambda b: (b, 0, 0),
                               memory_space=pltpu.SMEM),
        out_shape=jax.ShapeDtypeStruct((B, 3, G), jnp.float32),
        compiler_params=pltpu.CompilerParams(
            dimension_semantics=("parallel",)),
        interpret=interpret,
    )(xflat, xt)


CPB = 8


def _d2_body(centers_ref, xt_ref, d2_ref, t_ref, rm_ref):
    x = xt_ref[0, 0]
    y = xt_ref[0, 1]
    z = xt_ref[0, 2]
    ident = (lax.broadcasted_iota(jnp.int32, (NR, NR), 0)
             == lax.broadcasted_iota(jnp.int32, (NR, NR), 1)
             ).astype(jnp.float32)
    neg_inf = jnp.float32(-jnp.inf)
    for c in range(CPB):
        cx = centers_ref[0, c, 0]
        cy = centers_ref[0, c, 1]
        cz = centers_ref[0, c, 2]
        dx = cx - x
        dy = cy - y
        dz = cz - z
        d2 = dx * dx + dy * dy + dz * dz
        d2_ref[c] = d2
        rm = jnp.min(d2, axis=1, keepdims=True)
        rmt = lax.dot_general(rm, ident, (((0,), (0,)), ((), ())),
                              precision=lax.Precision.HIGHEST)
        rank = jnp.sum((rmt < rm).astype(jnp.int32), axis=1, keepdims=True)
        t = jnp.max(jnp.where(rank < M, rm, neg_inf))
        t_ref[0, 0, c] = t
        rm_ref[c] = rmt


def _d2_call(centers, xt, interpret=False):
    return pl.pallas_call(
        _d2_body,
        grid=(B, G // CPB),
        in_specs=[
            pl.BlockSpec((1, CPB, 3), lambda b, j: (b, j, 0),
                         memory_space=pltpu.SMEM),
            pl.BlockSpec((1, 3, NR, NC), lambda b, j: (b, 0, 0, 0)),
        ],
        out_specs=[
            pl.BlockSpec((CPB, NR, NC), lambda b, j: (b * (G // CPB) + j, 0, 0)),
            pl.BlockSpec((1, 1, CPB), lambda b, j: (b * (G // CPB) + j, 0, 0),
                         memory_space=pltpu.SMEM),
            pl.BlockSpec((CPB, 1, NR), lambda b, j: (b * (G // CPB) + j, 0, 0)),
        ],
        out_shape=[
            jax.ShapeDtypeStruct((B * G, NR, NC), jnp.float32),
            jax.ShapeDtypeStruct((B * G // CPB, 1, CPB), jnp.float32),
            jax.ShapeDtypeStruct((B * G, 1, NR), jnp.float32),
        ],
        compiler_params=pltpu.CompilerParams(
            dimension_semantics=("parallel", "parallel")),
        interpret=interpret,
    )(centers, xt)


NW = 32
WPB = NW // B
RPW = G // WPB
EPW = RPW * M
CAP = 1024


def _sel_body(xt_hbm, ct_hbm, t_hbm, d2_hbm, rm_hbm, out_hbm,
              xv, yv, zv, cxv, cyv, czv, tv, rmv, blist, dv,
              candv, candi, ov):
    w = lax.axis_index("s") * 2 + lax.axis_index("c")
    b = w // WPB
    r = w % WPB
    pltpu.sync_copy(xt_hbm.at[pl.ds(b * 3 * N, N)], xv)
    pltpu.sync_copy(xt_hbm.at[pl.ds((b * 3 + 1) * N, N)], yv)
    pltpu.sync_copy(xt_hbm.at[pl.ds((b * 3 + 2) * N, N)], zv)
    pltpu.sync_copy(ct_hbm.at[pl.ds(b * 3 * G, G)], cxv)
    pltpu.sync_copy(ct_hbm.at[pl.ds((b * 3 + 1) * G, G)], cyv)
    pltpu.sync_copy(ct_hbm.at[pl.ds((b * 3 + 2) * G, G)], czv)
    row0 = b * G + r * RPW
    pltpu.sync_copy(t_hbm.at[pl.ds(row0, RPW)], tv)

    lane = lax.iota(jnp.int32, 16)
    inf16 = jnp.full((16,), jnp.inf, dtype=jnp.float32)
    big16 = jnp.full((16,), _BIG, dtype=jnp.int32)

    def row_body(q, _):
        pltpu.sync_copy(d2_hbm.at[pl.ds((row0 + q) * N, N)], dv)
        pltpu.sync_copy(rm_hbm.at[pl.ds((row0 + q) * NR, NR)], rmv)
        tsv = plsc.load_gather(tv, [jnp.full((16,), q, dtype=jnp.int32)])

        def fchunk(t, off):
            rv = rmv[pl.ds(t * 16, 16)]
            mask = rv <= tsv
            plsc.store_compressed(blist.at[pl.ds(off, 16)],
                                  t * 16 + lane, mask=mask)
            return off + jnp.sum(mask.astype(jnp.int32))

        nb = lax.fori_loop(0, NR // 16, fchunk, jnp.int32(0))

        def rchunk(u, carry):
            off, tcnt = carry
            rvec = plsc.load_gather(
                blist, [jnp.full((16,), u, dtype=jnp.int32)])
            base = jnp.sum(jnp.where(lane == 0, rvec, 0)) * NC

            def ichunk(t, c2):
                off2, tc2 = c2
                v = dv[pl.ds(base + t * 16, 16)]
                mask = v <= tsv
                plsc.store_compressed(candv.at[pl.ds(off2, 16)], v,
                                      mask=mask)
                plsc.store_compressed(candi.at[pl.ds(off2, 16)],
                                      base + t * 16 + lane, mask=mask)
                cnt = jnp.sum(mask.astype(jnp.int32))
                return jnp.minimum(off2 + cnt, CAP - 16), tc2 + cnt

            return lax.fori_loop(0, NC // 16, ichunk, (off, tcnt))

        off, tcnt = lax.fori_loop(0, nb, rchunk,
                                  (jnp.int32(0), jnp.int32(0)))
        candv[pl.ds(off, 16)] = inf16
        candi[pl.ds(off, 16)] = big16
        nv = off // 16 + 1

        def run_select(load_pair, nvec):
            def select(k, carry):
                mprev, iprev, sel0, sel1 = carry

                def pass1(t, mv):
                    cv, ci = load_pair(t)
                    elig = (cv > mprev) | ((cv == mprev) & (ci > iprev))
                    return jnp.minimum(mv, jnp.where(elig, cv, inf16))

                m = jnp.min(lax.fori_loop(0, nvec, pass1, inf16))

                def pass2(t, iv):
                    cv, ci = load_pair(t)
                    elig = (cv == m) & ((cv > mprev) | (ci > iprev))
                    return jnp.minimum(iv, jnp.where(elig, ci, big16))

                i = jnp.min(lax.fori_loop(0, nvec, pass2, big16))
                sel0 = jnp.where(lane == k, i, sel0)
                sel1 = jnp.where(lane == (k - 16), i, sel1)
                return m, i, sel0, sel1

            zero16 = jnp.zeros((16,), dtype=jnp.int32)
            _, _, sel0, sel1 = lax.fori_loop(
                0, M, select, (jnp.float32(-jnp.inf), jnp.int32(-1),
                               zero16, zero16))
            return sel0, sel1

        def load_cand(t):
            return candv[pl.ds(t * 16, 16)], candi[pl.ds(t * 16, 16)]

        def load_full(t):
            return dv[pl.ds(t * 16, 16)], t * 16 + lane

        sel0, sel1 = lax.cond(
            tcnt <= CAP - 16,
            lambda: run_select(load_cand, nv),
            lambda: run_select(load_full, jnp.int32(N // 16)))

        gl = jnp.full((16,), r * RPW + q, dtype=jnp.int32)
        hx = plsc.load_gather(cxv, [gl])
        hy = plsc.load_gather(cyv, [gl])
        hz = plsc.load_gather(czv, [gl])
        o = q * M
        ov[pl.ds(o, 16)] = plsc.load_gather(xv, [sel0]) - hx
        ov[pl.ds(o + 16, 16)] = plsc.load_gather(xv, [sel1]) - hx
        ov[pl.ds(EPW + o, 16)] = plsc.load_gather(yv, [sel0]) - hy
        ov[pl.ds(EPW + o + 16, 16)] = plsc.load_gather(yv, [sel1]) - hy
        ov[pl.ds(2 * EPW + o, 16)] = plsc.load_gather(zv, [sel0]) - hz
        ov[pl.ds(2 * EPW + o + 16, 16)] = plsc.load_gather(zv, [sel1]) - hz
        return _

    lax.fori_loop(0, RPW, row_body, 0)
    off_out = b * G * M + r * EPW
    pltpu.sync_copy(ov.at[pl.ds(0, EPW)],
                    out_hbm.at[pl.ds(0 * B * G * M + off_out, EPW)])
    pltpu.sync_copy(ov.at[pl.ds(EPW, EPW)],
                    out_hbm.at[pl.ds(1 * B * G * M + off_out, EPW)])
    pltpu.sync_copy(ov.at[pl.ds(2 * EPW, EPW)],
                    out_hbm.at[pl.ds(2 * B * G * M + off_out, EPW)])


def _sel_call(xt_flat, ct_flat, t_flat, d2_flat, rm_flat):
    mesh = plsc.VectorSubcoreMesh(core_axis_name="c", subcore_axis_name="s")
    kfn = pl.kernel(
        _sel_body,
        mesh=mesh,
        compiler_params=pltpu.CompilerParams(needs_layout_passes=False),
        out_type=jax.ShapeDtypeStruct((3 * B * G * M,), jnp.float32),
        scratch_types=[
            pltpu.VMEM((N,), jnp.float32),
            pltpu.VMEM((N,), jnp.float32),
            pltpu.VMEM((N,), jnp.float32),
            pltpu.VMEM((G,), jnp.float32),
            pltpu.VMEM((G,), jnp.float32),
            pltpu.VMEM((G,), jnp.float32),
            pltpu.VMEM((RPW,), jnp.float32),
            pltpu.VMEM((NR,), jnp.float32),
            pltpu.VMEM((NR + 16,), jnp.int32),
            pltpu.VMEM((N,), jnp.float32),
            pltpu.VMEM((CAP,), jnp.float32),
            pltpu.VMEM((CAP,), jnp.int32),
            pltpu.VMEM((3 * EPW,), jnp.float32),
        ],
    )
    return kfn(xt_flat, ct_flat, t_flat, d2_flat, rm_flat)


@jax.jit
def kernel(xyz):
    xt = jnp.transpose(xyz, (0, 2, 1)).reshape(B, 3, NR, NC)
    cs = _fps_call(xt.reshape(B, 3, N), xt)
    centers = jnp.transpose(cs, (0, 2, 1))
    d2, tthr, rmt = _d2_call(centers, xt)
    out3 = _sel_call(
        xt.reshape(B * 3 * N),
        cs.reshape(B * 3 * G),
        tthr.reshape(B * G),
        d2.reshape(B * G * N),
        rmt.reshape(B * G * NR),
    )
    neighborhood = jnp.transpose(out3.reshape(3, B, G, M), (1, 2, 3, 0))
    return neighborhood, centers


# speedup vs baseline: 14.4106x; 1.0027x over previous
import jax
import jax.numpy as jnp
from jax import lax
from jax.experimental import pallas as pl
from jax.experimental.pallas import tpu as pltpu
from jax.experimental.pallas import tpu_sc as plsc

B = 4
N = 16384
G = 512
M = 32
NR = 128
NC = 128

_BIG = 1 << 30


def _fps_body(xs_ref, xt_ref, cs_ref):
    fiota = (lax.broadcasted_iota(jnp.int32, (NR, NC), 0) * NC
             + lax.broadcasted_iota(jnp.int32, (NR, NC), 1))
    dists0 = jnp.full((NR, NC), jnp.inf, dtype=jnp.float32)
    x = xt_ref[0, 0]
    y = xt_ref[0, 1]
    z = xt_ref[0, 2]

    def step(g, carry):
        px, py, pz, dists = carry
        cs_ref[0, 0, g] = px
        cs_ref[0, 1, g] = py
        cs_ref[0, 2, g] = pz
        dx = x - px
        dy = y - py
        dz = z - pz
        d = dx * dx + dy * dy + dz * dz
        dists = jnp.minimum(dists, d)
        m = jnp.max(dists)
        sel = jnp.where(dists == m, fiota, _BIG)
        i = jnp.min(sel)
        return xs_ref[0, 0, i], xs_ref[0, 1, i], xs_ref[0, 2, i], dists

    lax.fori_loop(0, G, step,
                  (xs_ref[0, 0, 0], xs_ref[0, 1, 0], xs_ref[0, 2, 0],
                   dists0))


def _fps_call(xflat, xt, interpret=False):
    return pl.pallas_call(
        _fps_body,
        grid=(B,),
        in_specs=[
            pl.BlockSpec((1, 3, N), lambda b: (b, 0, 0),
                         memory_space=pltpu.SMEM),
            pl.BlockSpec((1, 3, NR, NC), lambda b: (b, 0, 0, 0)),
        ],
        out_specs=pl.BlockSpec((1, 3, G), lambda b: (b, 0, 0),
                               memory_space=pltpu.SMEM),
        out_shape=jax.ShapeDtypeStruct((B, 3, G), jnp.float32),
        compiler_params=pltpu.CompilerParams(
            dimension_semantics=("arbitrary",)),
        interpret=interpret,
    )(xflat, xt)


CPB = 8


def _d2_body(centers_ref, xt_ref, d2_ref, t_ref, rm_ref):
    x = xt_ref[0, 0]
    y = xt_ref[0, 1]
    z = xt_ref[0, 2]
    ident = (lax.broadcasted_iota(jnp.int32, (NR, NR), 0)
             == lax.broadcasted_iota(jnp.int32, (NR, NR), 1)
             ).astype(jnp.float32)
    neg_inf = jnp.float32(-jnp.inf)
    for c in range(CPB):
        cx = centers_ref[0, c, 0]
        cy = centers_ref[0, c, 1]
        cz = centers_ref[0, c, 2]
        dx = cx - x
        dy = cy - y
        dz = cz - z
        d2 = dx * dx + dy * dy + dz * dz
        d2_ref[c] = d2
        rm = jnp.min(d2, axis=1, keepdims=True)
        rmt = lax.dot_general(rm, ident, (((0,), (0,)), ((), ())),
                              precision=lax.Precision.HIGHEST)
        rank = jnp.sum((rmt < rm).astype(jnp.int32), axis=1, keepdims=True)
        t = jnp.max(jnp.where(rank < M, rm, neg_inf))
        t_ref[0, 0, c] = t
        rm_ref[c] = rmt


def _d2_call(centers, xt, interpret=False):
    return pl.pallas_call(
        _d2_body,
        grid=(B, G // CPB),
        in_specs=[
            pl.BlockSpec((1, CPB, 3), lambda b, j: (b, j, 0),
                         memory_space=pltpu.SMEM),
            pl.BlockSpec((1, 3, NR, NC), lambda b, j: (b, 0, 0, 0)),
        ],
        out_specs=[
            pl.BlockSpec((CPB, NR, NC), lambda b, j: (b * (G // CPB) + j, 0, 0)),
            pl.BlockSpec((1, 1, CPB), lambda b, j: (b * (G // CPB) + j, 0, 0),
                         memory_space=pltpu.SMEM),
            pl.BlockSpec((CPB, 1, NR), lambda b, j: (b * (G // CPB) + j, 0, 0)),
        ],
        out_shape=[
            jax.ShapeDtypeStruct((B * G, NR, NC), jnp.float32),
            jax.ShapeDtypeStruct((B * G // CPB, 1, CPB), jnp.float32),
            jax.ShapeDtypeStruct((B * G, 1, NR), jnp.float32),
        ],
        compiler_params=pltpu.CompilerParams(
            dimension_semantics=("parallel", "parallel")),
        interpret=interpret,
    )(centers, xt)


NW = 32
WPB = NW // B
RPW = G // WPB
EPW = RPW * M
CAP = 1024


def _sel_body(xt_hbm, ct_hbm, t_hbm, d2_hbm, rm_hbm, out_hbm,
              xv, yv, zv, cxv, cyv, czv, tv, rmv, blist, dv,
              candv, candi, ov):
    w = lax.axis_index("s") * 2 + lax.axis_index("c")
    b = w // WPB
    r = w % WPB
    pltpu.sync_copy(xt_hbm.at[pl.ds(b * 3 * N, N)], xv)
    pltpu.sync_copy(xt_hbm.at[pl.ds((b * 3 + 1) * N, N)], yv)
    pltpu.sync_copy(xt_hbm.at[pl.ds((b * 3 + 2) * N, N)], zv)
    pltpu.sync_copy(ct_hbm.at[pl.ds(b * 3 * G, G)], cxv)
    pltpu.sync_copy(ct_hbm.at[pl.ds((b * 3 + 1) * G, G)], cyv)
    pltpu.sync_copy(ct_hbm.at[pl.ds((b * 3 + 2) * G, G)], czv)
    row0 = b * G + r * RPW
    pltpu.sync_copy(t_hbm.at[pl.ds(row0, RPW)], tv)

    lane = lax.iota(jnp.int32, 16)
    inf16 = jnp.full((16,), jnp.inf, dtype=jnp.float32)
    big16 = jnp.full((16,), _BIG, dtype=jnp.int32)

    def row_body(q, _):
        pltpu.sync_copy(d2_hbm.at[pl.ds((row0 + q) * N, N)], dv)
        pltpu.sync_copy(rm_hbm.at[pl.ds((row0 + q) * NR, NR)], rmv)
        tsv = plsc.load_gather(tv, [jnp.full((16,), q, dtype=jnp.int32)])

        def fchunk(t, off):
            rv = rmv[pl.ds(t * 16, 16)]
            mask = rv <= tsv
            plsc.store_compressed(blist.at[pl.ds(off, 16)],
                                  t * 16 + lane, mask=mask)
            return off + jnp.sum(mask.astype(jnp.int32))

        nb = lax.fori_loop(0, NR // 16, fchunk, jnp.int32(0))

        def rchunk(u, carry):
            off, tcnt = carry
            rvec = plsc.load_gather(
                blist, [jnp.full((16,), u, dtype=jnp.int32)])
            base = jnp.sum(jnp.where(lane == 0, rvec, 0)) * NC

            def ichunk(t, c2):
                off2, tc2 = c2
                v = dv[pl.ds(base + t * 16, 16)]
                mask = v <= tsv
                plsc.store_compressed(candv.at[pl.ds(off2, 16)], v,
                                      mask=mask)
                plsc.store_compressed(candi.at[pl.ds(off2, 16)],
                                      base + t * 16 + lane, mask=mask)
                cnt = jnp.sum(mask.astype(jnp.int32))
                return jnp.minimum(off2 + cnt, CAP - 16), tc2 + cnt

            return lax.fori_loop(0, NC // 16, ichunk, (off, tcnt))

        off, tcnt = lax.fori_loop(0, nb, rchunk,
                                  (jnp.int32(0), jnp.int32(0)))
        candv[pl.ds(off, 16)] = inf16
        candi[pl.ds(off, 16)] = big16
        nv = off // 16 + 1

        def run_select(load_pair, nvec):
            def select(k, carry):
                mprev, iprev, sel0, sel1 = carry

                def pass1(t, mv):
                    cv, ci = load_pair(t)
                    elig = (cv > mprev) | ((cv == mprev) & (ci > iprev))
                    return jnp.minimum(mv, jnp.where(elig, cv, inf16))

                m = jnp.min(lax.fori_loop(0, nvec, pass1, inf16))

                def pass2(t, iv):
                    cv, ci = load_pair(t)
                    elig = (cv == m) & ((cv > mprev) | (ci > iprev))
                    return jnp.minimum(iv, jnp.where(elig, ci, big16))

                i = jnp.min(lax.fori_loop(0, nvec, pass2, big16))
                sel0 = jnp.where(lane == k, i, sel0)
                sel1 = jnp.where(lane == (k - 16), i, sel1)
                return m, i, sel0, sel1

            zero16 = jnp.zeros((16,), dtype=jnp.int32)
            _, _, sel0, sel1 = lax.fori_loop(
                0, M, select, (jnp.float32(-jnp.inf), jnp.int32(-1),
                               zero16, zero16))
            return sel0, sel1

        def load_cand(t):
            return candv[pl.ds(t * 16, 16)], candi[pl.ds(t * 16, 16)]

        def load_full(t):
            return dv[pl.ds(t * 16, 16)], t * 16 + lane

        sel0, sel1 = lax.cond(
            tcnt <= CAP - 16,
            lambda: run_select(load_cand, nv),
            lambda: run_select(load_full, jnp.int32(N // 16)))

        gl = jnp.full((16,), r * RPW + q, dtype=jnp.int32)
        hx = plsc.load_gather(cxv, [gl])
        hy = plsc.load_gather(cyv, [gl])
        hz = plsc.load_gather(czv, [gl])
        o = q * M
        ov[pl.ds(o, 16)] = plsc.load_gather(xv, [sel0]) - hx
        ov[pl.ds(o + 16, 16)] = plsc.load_gather(xv, [sel1]) - hx
        ov[pl.ds(EPW + o, 16)] = plsc.load_gather(yv, [sel0]) - hy
        ov[pl.ds(EPW + o + 16, 16)] = plsc.load_gather(yv, [sel1]) - hy
        ov[pl.ds(2 * EPW + o, 16)] = plsc.load_gather(zv, [sel0]) - hz
        ov[pl.ds(2 * EPW + o + 16, 16)] = plsc.load_gather(zv, [sel1]) - hz
        return _

    lax.fori_loop(0, RPW, row_body, 0)
    off_out = b * G * M + r * EPW
    pltpu.sync_copy(ov.at[pl.ds(0, EPW)],
                    out_hbm.at[pl.ds(0 * B * G * M + off_out, EPW)])
    pltpu.sync_copy(ov.at[pl.ds(EPW, EPW)],
                    out_hbm.at[pl.ds(1 * B * G * M + off_out, EPW)])
    pltpu.sync_copy(ov.at[pl.ds(2 * EPW, EPW)],
                    out_hbm.at[pl.ds(2 * B * G * M + off_out, EPW)])


def _sel_call(xt_flat, ct_flat, t_flat, d2_flat, rm_flat):
    mesh = plsc.VectorSubcoreMesh(core_axis_name="c", subcore_axis_name="s")
    kfn = pl.kernel(
        _sel_body,
        mesh=mesh,
        compiler_params=pltpu.CompilerParams(needs_layout_passes=False),
        out_type=jax.ShapeDtypeStruct((3 * B * G * M,), jnp.float32),
        scratch_types=[
            pltpu.VMEM((N,), jnp.float32),
            pltpu.VMEM((N,), jnp.float32),
            pltpu.VMEM((N,), jnp.float32),
            pltpu.VMEM((G,), jnp.float32),
            pltpu.VMEM((G,), jnp.float32),
            pltpu.VMEM((G,), jnp.float32),
            pltpu.VMEM((RPW,), jnp.float32),
            pltpu.VMEM((NR,), jnp.float32),
            pltpu.VMEM((NR + 16,), jnp.int32),
            pltpu.VMEM((N,), jnp.float32),
            pltpu.VMEM((CAP,), jnp.float32),
            pltpu.VMEM((CAP,), jnp.int32),
            pltpu.VMEM((3 * EPW,), jnp.float32),
        ],
    )
    return kfn(xt_flat, ct_flat, t_flat, d2_flat, rm_flat)


@jax.jit
def kernel(xyz):
    xt = jnp.transpose(xyz, (0, 2, 1)).reshape(B, 3, NR, NC)
    cs = _fps_call(xt.reshape(B, 3, N), xt)
    centers = jnp.transpose(cs, (0, 2, 1))
    d2, tthr, rmt = _d2_call(centers, xt)
    out3 = _sel_call(
        xt.reshape(B * 3 * N),
        cs.reshape(B * 3 * G),
        tthr.reshape(B * G),
        d2.reshape(B * G * N),
        rmt.reshape(B * G * NR),
    )
    neighborhood = jnp.transpose(out3.reshape(3, B, G, M), (1, 2, 3, 0))
    return neighborhood, centers


# speedup vs baseline: 15.6905x; 1.0888x over previous
import jax
import jax.numpy as jnp
from jax import lax
from jax.experimental import pallas as pl
from jax.experimental.pallas import tpu as pltpu
from jax.experimental.pallas import tpu_sc as plsc

B = 4
N = 16384
G = 512
M = 32
NR = 128
NC = 128

_BIG = 1 << 30


BPP = 2


def _fps_body(xs_ref, xt_ref, cs_ref):
    fiota = (lax.broadcasted_iota(jnp.int32, (NR, NC), 0) * NC
             + lax.broadcasted_iota(jnp.int32, (NR, NC), 1))
    dists0 = jnp.full((NR, NC), jnp.inf, dtype=jnp.float32)

    init = []
    for b in range(BPP):
        init += [xs_ref[b, 0, 0], xs_ref[b, 1, 0], xs_ref[b, 2, 0], dists0]

    def step(g, carry):
        out = []
        for b in range(BPP):
            px, py, pz, dists = carry[4 * b:4 * b + 4]
            cs_ref[b, 0, g] = px
            cs_ref[b, 1, g] = py
            cs_ref[b, 2, g] = pz
            dx = xt_ref[b, 0] - px
            dy = xt_ref[b, 1] - py
            dz = xt_ref[b, 2] - pz
            d = dx * dx + dy * dy + dz * dz
            dists = jnp.minimum(dists, d)
            m = jnp.max(dists)
            sel = jnp.where(dists == m, fiota, _BIG)
            i = jnp.min(sel)
            out += [xs_ref[b, 0, i], xs_ref[b, 1, i], xs_ref[b, 2, i],
                    dists]
        return tuple(out)

    lax.fori_loop(0, G, step, tuple(init))


def _fps_call(xflat, xt, interpret=False):
    call = pl.pallas_call(
        _fps_body,
        grid=(1,),
        in_specs=[
            pl.BlockSpec((BPP, 3, N), lambda i: (0, 0, 0),
                         memory_space=pltpu.SMEM),
            pl.BlockSpec((BPP, 3, NR, NC), lambda i: (0, 0, 0, 0)),
        ],
        out_specs=pl.BlockSpec((BPP, 3, G), lambda i: (0, 0, 0),
                               memory_space=pltpu.SMEM),
        out_shape=jax.ShapeDtypeStruct((BPP, 3, G), jnp.float32),
        interpret=interpret,
    )
    return jnp.concatenate(
        [call(xflat[p:p + BPP], xt[p:p + BPP])
         for p in range(0, B, BPP)], axis=0)


CPB = 8


def _d2_body(centers_ref, xt_ref, d2_ref, t_ref, rm_ref):
    x = xt_ref[0, 0]
    y = xt_ref[0, 1]
    z = xt_ref[0, 2]
    ident = (lax.broadcasted_iota(jnp.int32, (NR, NR), 0)
             == lax.broadcasted_iota(jnp.int32, (NR, NR), 1)
             ).astype(jnp.float32)
    neg_inf = jnp.float32(-jnp.inf)
    for c in range(CPB):
        cx = centers_ref[0, c, 0]
        cy = centers_ref[0, c, 1]
        cz = centers_ref[0, c, 2]
        dx = cx - x
        dy = cy - y
        dz = cz - z
        d2 = dx * dx + dy * dy + dz * dz
        d2_ref[c] = d2
        rm = jnp.min(d2, axis=1, keepdims=True)
        rmt = lax.dot_general(rm, ident, (((0,), (0,)), ((), ())),
                              precision=lax.Precision.HIGHEST)
        rank = jnp.sum((rmt < rm).astype(jnp.int32), axis=1, keepdims=True)
        t = jnp.max(jnp.where(rank < M, rm, neg_inf))
        t_ref[0, 0, c] = t
        rm_ref[c] = rmt


def _d2_call(centers, xt, interpret=False):
    return pl.pallas_call(
        _d2_body,
        grid=(B, G // CPB),
        in_specs=[
            pl.BlockSpec((1, CPB, 3), lambda b, j: (b, j, 0),
                         memory_space=pltpu.SMEM),
            pl.BlockSpec((1, 3, NR, NC), lambda b, j: (b, 0, 0, 0)),
        ],
        out_specs=[
            pl.BlockSpec((CPB, NR, NC), lambda b, j: (b * (G // CPB) + j, 0, 0)),
            pl.BlockSpec((1, 1, CPB), lambda b, j: (b * (G // CPB) + j, 0, 0),
                         memory_space=pltpu.SMEM),
            pl.BlockSpec((CPB, 1, NR), lambda b, j: (b * (G // CPB) + j, 0, 0)),
        ],
        out_shape=[
            jax.ShapeDtypeStruct((B * G, NR, NC), jnp.float32),
            jax.ShapeDtypeStruct((B * G // CPB, 1, CPB), jnp.float32),
            jax.ShapeDtypeStruct((B * G, 1, NR), jnp.float32),
        ],
        compiler_params=pltpu.CompilerParams(
            dimension_semantics=("parallel", "parallel")),
        interpret=interpret,
    )(centers, xt)


NW = 32
WPB = NW // B
RPW = G // WPB
EPW = RPW * M
CAP = 1024


def _sel_body(xt_hbm, ct_hbm, t_hbm, d2_hbm, rm_hbm, out_hbm,
              xv, yv, zv, cxv, cyv, czv, tv, rmv, blist, dv,
              candv, candi, ov):
    w = lax.axis_index("s") * 2 + lax.axis_index("c")
    b = w // WPB
    r = w % WPB
    pltpu.sync_copy(xt_hbm.at[pl.ds(b * 3 * N, N)], xv)
    pltpu.sync_copy(xt_hbm.at[pl.ds((b * 3 + 1) * N, N)], yv)
    pltpu.sync_copy(xt_hbm.at[pl.ds((b * 3 + 2) * N, N)], zv)
    pltpu.sync_copy(ct_hbm.at[pl.ds(b * 3 * G, G)], cxv)
    pltpu.sync_copy(ct_hbm.at[pl.ds((b * 3 + 1) * G, G)], cyv)
    pltpu.sync_copy(ct_hbm.at[pl.ds((b * 3 + 2) * G, G)], czv)
    row0 = b * G + r * RPW
    pltpu.sync_copy(t_hbm.at[pl.ds(row0, RPW)], tv)

    lane = lax.iota(jnp.int32, 16)
    inf16 = jnp.full((16,), jnp.inf, dtype=jnp.float32)
    big16 = jnp.full((16,), _BIG, dtype=jnp.int32)

    def row_body(q, _):
        pltpu.sync_copy(d2_hbm.at[pl.ds((row0 + q) * N, N)], dv)
        pltpu.sync_copy(rm_hbm.at[pl.ds((row0 + q) * NR, NR)], rmv)
        tsv = plsc.load_gather(tv, [jnp.full((16,), q, dtype=jnp.int32)])

        def fchunk(t, off):
            rv = rmv[pl.ds(t * 16, 16)]
            mask = rv <= tsv
            plsc.store_compressed(blist.at[pl.ds(off, 16)],
                                  t * 16 + lane, mask=mask)
            return off + jnp.sum(mask.astype(jnp.int32))

        nb = lax.fori_loop(0, NR // 16, fchunk, jnp.int32(0))

        def rchunk(u, carry):
            off, tcnt = carry
            rvec = plsc.load_gather(
                blist, [jnp.full((16,), u, dtype=jnp.int32)])
            base = jnp.sum(jnp.where(lane == 0, rvec, 0)) * NC

            def ichunk(t, c2):
                off2, tc2 = c2
                v = dv[pl.ds(base + t * 16, 16)]
                mask = v <= tsv
                plsc.store_compressed(candv.at[pl.ds(off2, 16)], v,
                                      mask=mask)
                plsc.store_compressed(candi.at[pl.ds(off2, 16)],
                                      base + t * 16 + lane, mask=mask)
                cnt = jnp.sum(mask.astype(jnp.int32))
                return jnp.minimum(off2 + cnt, CAP - 16), tc2 + cnt

            return lax.fori_loop(0, NC // 16, ichunk, (off, tcnt))

        off, tcnt = lax.fori_loop(0, nb, rchunk,
                                  (jnp.int32(0), jnp.int32(0)))
        candv[pl.ds(off, 16)] = inf16
        candi[pl.ds(off, 16)] = big16
        nv = off // 16 + 1

        def run_select(load_pair, nvec):
            def select(k, carry):
                mprev, iprev, sel0, sel1 = carry

                def pass1(t, mv):
                    cv, ci = load_pair(t)
                    elig = (cv > mprev) | ((cv == mprev) & (ci > iprev))
                    return jnp.minimum(mv, jnp.where(elig, cv, inf16))

                m = jnp.min(lax.fori_loop(0, nvec, pass1, inf16))

                def pass2(t, iv):
                    cv, ci = load_pair(t)
                    elig = (cv == m) & ((cv > mprev) | (ci > iprev))
                    return jnp.minimum(iv, jnp.where(elig, ci, big16))

                i = jnp.min(lax.fori_loop(0, nvec, pass2, big16))
                sel0 = jnp.where(lane == k, i, sel0)
                sel1 = jnp.where(lane == (k - 16), i, sel1)
                return m, i, sel0, sel1

            zero16 = jnp.zeros((16,), dtype=jnp.int32)
            _, _, sel0, sel1 = lax.fori_loop(
                0, M, select, (jnp.float32(-jnp.inf), jnp.int32(-1),
                               zero16, zero16))
            return sel0, sel1

        def load_cand(t):
            return candv[pl.ds(t * 16, 16)], candi[pl.ds(t * 16, 16)]

        def load_full(t):
            return dv[pl.ds(t * 16, 16)], t * 16 + lane

        sel0, sel1 = lax.cond(
            tcnt <= CAP - 16,
            lambda: run_select(load_cand, nv),
            lambda: run_select(load_full, jnp.int32(N // 16)))

        gl = jnp.full((16,), r * RPW + q, dtype=jnp.int32)
        hx = plsc.load_gather(cxv, [gl])
        hy = plsc.load_gather(cyv, [gl])
        hz = plsc.load_gather(czv, [gl])
        o = q * M
        ov[pl.ds(o, 16)] = plsc.load_gather(xv, [sel0]) - hx
        ov[pl.ds(o + 16, 16)] = plsc.load_gather(xv, [sel1]) - hx
        ov[pl.ds(EPW + o, 16)] = plsc.load_gather(yv, [sel0]) - hy
        ov[pl.ds(EPW + o + 16, 16)] = plsc.load_gather(yv, [sel1]) - hy
        ov[pl.ds(2 * EPW + o, 16)] = plsc.load_gather(zv, [sel0]) - hz
        ov[pl.ds(2 * EPW + o + 16, 16)] = plsc.load_gather(zv, [sel1]) - hz
        return _

    lax.fori_loop(0, RPW, row_body, 0)
    off_out = b * G * M + r * EPW
    pltpu.sync_copy(ov.at[pl.ds(0, EPW)],
                    out_hbm.at[pl.ds(0 * B * G * M + off_out, EPW)])
    pltpu.sync_copy(ov.at[pl.ds(EPW, EPW)],
                    out_hbm.at[pl.ds(1 * B * G * M + off_out, EPW)])
    pltpu.sync_copy(ov.at[pl.ds(2 * EPW, EPW)],
                    out_hbm.at[pl.ds(2 * B * G * M + off_out, EPW)])


def _sel_call(xt_flat, ct_flat, t_flat, d2_flat, rm_flat):
    mesh = plsc.VectorSubcoreMesh(core_axis_name="c", subcore_axis_name="s")
    kfn = pl.kernel(
        _sel_body,
        mesh=mesh,
        compiler_params=pltpu.CompilerParams(needs_layout_passes=False),
        out_type=jax.ShapeDtypeStruct((3 * B * G * M,), jnp.float32),
        scratch_types=[
            pltpu.VMEM((N,), jnp.float32),
            pltpu.VMEM((N,), jnp.float32),
            pltpu.VMEM((N,), jnp.float32),
            pltpu.VMEM((G,), jnp.float32),
            pltpu.VMEM((G,), jnp.float32),
            pltpu.VMEM((G,), jnp.float32),
            pltpu.VMEM((RPW,), jnp.float32),
            pltpu.VMEM((NR,), jnp.float32),
            pltpu.VMEM((NR + 16,), jnp.int32),
            pltpu.VMEM((N,), jnp.float32),
            pltpu.VMEM((CAP,), jnp.float32),
            pltpu.VMEM((CAP,), jnp.int32),
            pltpu.VMEM((3 * EPW,), jnp.float32),
        ],
    )
    return kfn(xt_flat, ct_flat, t_flat, d2_flat, rm_flat)


@jax.jit
def kernel(xyz):
    xt = jnp.transpose(xyz, (0, 2, 1)).reshape(B, 3, NR, NC)
    cs = _fps_call(xt.reshape(B, 3, N), xt)
    centers = jnp.transpose(cs, (0, 2, 1))
    d2, tthr, rmt = _d2_call(centers, xt)
    out3 = _sel_call(
        xt.reshape(B * 3 * N),
        cs.reshape(B * 3 * G),
        tthr.reshape(B * G),
        d2.reshape(B * G * N),
        rmt.reshape(B * G * NR),
    )
    neighborhood = jnp.transpose(out3.reshape(3, B, G, M), (1, 2, 3, 0))
    return neighborhood, centers
